# baseline (speedup 1.0000x reference)
import functools

import jax
import jax.numpy as jnp
from jax import lax
from jax.experimental import pallas as pl
from jax.experimental.pallas import tpu as pltpu
from jax.experimental.pallas import tpu_sc as plsc

L = 16
SUB = 8
PH = L
NBUF = 3


def _lane_shuffle(v, perm):
    dnums = lax.GatherDimensionNumbers(
        offset_dims=(), collapsed_slice_dims=(0,), start_index_map=(0,))
    return lax.gather(v, perm.reshape(L, 1), dnums, slice_sizes=(1,),
                      mode=lax.GatherScatterMode.PROMISE_IN_BOUNDS)


def _neg_sqrt(s):
    i = lax.bitcast_convert_type(s, jnp.int32)
    i = jnp.int32(0x5F3759DF) - lax.shift_right_logical(i, 1)
    y = lax.bitcast_convert_type(i, jnp.float32)
    half_s = s * jnp.float32(0.5)
    for _ in range(3):
        y = y * (jnp.float32(1.5) - half_s * y * y)
    return -(s * y)


def _make_kernel(B, D, NC, NS):
    NW = NC * NS
    b_w = B // NW
    n_phase = b_w // PH
    d_chunks = D // L

    mesh = plsc.VectorSubcoreMesh(core_axis_name="c", subcore_axis_name="s")

    tile_t = pltpu.VMEM((PH, SUB, D), jnp.float32)

    @functools.partial(
        pl.kernel,
        mesh=mesh,
        compiler_params=pltpu.CompilerParams(use_tc_tiling_on_sc=True),
        out_type=jax.ShapeDtypeStruct((B,), jnp.float32),
        scratch_types=[
            pltpu.VMEM((b_w,), jnp.int32),
            pltpu.VMEM((b_w,), jnp.int32),
            pltpu.VMEM((b_w,), jnp.int32),
            pltpu.VMEM((b_w,), jnp.int32),
            tile_t, tile_t, tile_t,
            pltpu.VMEM((PH, 2 * D), jnp.float32),
            pltpu.VMEM((PH, 2 * D), jnp.float32),
            pltpu.VMEM((PH, 2 * D), jnp.float32),
            tile_t, tile_t, tile_t,
            pltpu.VMEM((b_w,), jnp.float32),
            pltpu.SemaphoreType.DMA,
            pltpu.SemaphoreType.DMA,
            pltpu.SemaphoreType.DMA,
        ],
    )
    def kge_kernel(head_hbm, rel_hbm, tail_hbm, node3_hbm, rel2_hbm,
                   out_hbm, ho, ro, to, r2i, hA, hB, hC, rA, rB, rC,
                   tA, tB, tC, out_v, semA, semB, semC):
        wid = lax.axis_index("s") * NC + lax.axis_index("c")
        base = wid * b_w

        pltpu.sync_copy(head_hbm.at[pl.ds(base, b_w)], ho)
        pltpu.sync_copy(rel_hbm.at[pl.ds(base, b_w)], ro)
        pltpu.sync_copy(tail_hbm.at[pl.ds(base, b_w)], to)

        for s in range(b_w // L):
            cs = pl.ds(s * L, L)
            r2i[cs] = lax.shift_right_logical(ro[cs], 1)

        seven = jnp.int32(7)
        three = jnp.int32(3)
        lane_iota = lax.iota(jnp.int32, L)

        def issue(p, bufs, sem):
            hbuf, rbuf, tbuf = bufs
            pltpu.async_copy(
                rel2_hbm.at[r2i.at[pl.ds(p * PH, PH)]], rbuf, sem)
            hv = ho[pl.ds(p * PH, L)]
            tv = to[pl.ds(p * PH, L)]
            for r in range(L):
                pltpu.async_copy(
                    node3_hbm.at[lax.shift_right_logical(hv[r], three)],
                    hbuf.at[r], sem)
                pltpu.async_copy(
                    node3_hbm.at[lax.shift_right_logical(tv[r], three)],
                    tbuf.at[r], sem)

        def drain(bufs, sem):
            hbuf, rbuf, tbuf = bufs
            pltpu.make_async_copy(node3_hbm.at[pl.ds(0, PH)], hbuf, sem).wait()
            pltpu.make_async_copy(node3_hbm.at[pl.ds(0, PH)], tbuf, sem).wait()
            pltpu.make_async_copy(rel2_hbm.at[pl.ds(0, PH)], rbuf, sem).wait()

        def compute(p, bufs):
            hbuf, rbuf, tbuf = bufs
            rb = p * PH
            hv = ho[pl.ds(rb, L)]
            tv = to[pl.ds(rb, L)]
            rp = (ro[pl.ds(rb, L)] & jnp.int32(1)).astype(jnp.float32)
            tot = None
            for r in range(L):
                hsub = hv[r] & seven
                tsub = tv[r] & seven
                rm = _lane_shuffle(rp, jnp.full((L,), r, jnp.int32))
                acc = None
                for c in range(d_chunks):
                    cs = pl.ds(c * L, L)
                    hs = pl.ds(D + c * L, L)
                    rl = rbuf[r, cs]
                    rv = rl + rm * (rbuf[r, hs] - rl)
                    d = hbuf[r, hsub, cs] + rv - tbuf[r, tsub, cs]
                    acc = d * d if acc is None else acc + d * d
                for step in (8, 4, 2, 1):
                    acc = acc + _lane_shuffle(acc, lane_iota ^ step)
                sel = lane_iota == jnp.int32(r)
                tot = acc if r == 0 else jnp.where(sel, acc, tot)
            out_v[pl.ds(rb, L)] = _neg_sqrt(tot + jnp.float32(1e-12))

        bufs = [(hA, rA, tA), (hB, rB, tB), (hC, rC, tC)]
        sems = [semA, semB, semC]

        issue(0, bufs[0], sems[0])
        issue(1, bufs[1], sems[1])

        def body(k, carry):
            for j in range(NBUF):
                p = NBUF * k + j
                nxt = (j + 2) % NBUF
                drain(bufs[j], sems[j])

                @pl.when(p + 2 < n_phase)
                def _():
                    issue(p + 2, bufs[nxt], sems[nxt])

                compute(p, bufs[j])
            return carry

        lax.fori_loop(0, n_phase // NBUF, body, 0)
        for p in range(n_phase - n_phase % NBUF, n_phase):
            j = p % NBUF
            drain(bufs[j], sems[j])
            compute(p, bufs[j])
        pltpu.sync_copy(out_v, out_hbm.at[pl.ds(base, b_w)])

    return kge_kernel


def kernel(head_index, rel_type, tail_index, node_emb, rel_emb):
    B = head_index.shape[0]
    D = node_emb.shape[1]
    info = plsc.get_sparse_core_info()
    k = _make_kernel(B, D, info.num_cores, info.num_subcores)
    node3 = node_emb.reshape(-1, SUB, D)
    rel2 = rel_emb.reshape(-1, 2 * D)
    return k(head_index.astype(jnp.int32), rel_type.astype(jnp.int32),
             tail_index.astype(jnp.int32), node3, rel2)


# speedup vs baseline: 1.0018x; 1.0018x over previous
import functools

import jax
import jax.numpy as jnp
from jax import lax
from jax.experimental import pallas as pl
from jax.experimental.pallas import tpu as pltpu
from jax.experimental.pallas import tpu_sc as plsc

L = 16
SUB = 8


def _lane_shuffle(v, perm):
    dnums = lax.GatherDimensionNumbers(
        offset_dims=(), collapsed_slice_dims=(0,), start_index_map=(0,))
    return lax.gather(v, perm.reshape(L, 1), dnums, slice_sizes=(1,),
                      mode=lax.GatherScatterMode.PROMISE_IN_BOUNDS)


def _neg_sqrt(s):
    i = lax.bitcast_convert_type(s, jnp.int32)
    i = jnp.int32(0x5F3759DF) - lax.shift_right_logical(i, 1)
    y = lax.bitcast_convert_type(i, jnp.float32)
    half_s = s * jnp.float32(0.5)
    for _ in range(3):
        y = y * (jnp.float32(1.5) - half_s * y * y)
    return -(s * y)


def _make_kernel(B, D, NC, NS):
    NW = NC * NS
    b_w = B // NW
    n_phase = b_w // L
    d_chunks = D // L

    mesh = plsc.VectorSubcoreMesh(core_axis_name="c", subcore_axis_name="s")

    tile_t = pltpu.VMEM((L, SUB, D), jnp.float32)

    @functools.partial(
        pl.kernel,
        mesh=mesh,
        compiler_params=pltpu.CompilerParams(use_tc_tiling_on_sc=True),
        out_type=jax.ShapeDtypeStruct((B,), jnp.float32),
        scratch_types=[
            pltpu.VMEM((b_w,), jnp.int32),
            pltpu.VMEM((b_w,), jnp.int32),
            pltpu.VMEM((b_w,), jnp.int32),
            pltpu.VMEM((b_w,), jnp.int32),
            tile_t, tile_t,
            pltpu.VMEM((L, 2 * D), jnp.float32),
            pltpu.VMEM((L, 2 * D), jnp.float32),
            tile_t, tile_t,
            pltpu.VMEM((b_w,), jnp.float32),
            pltpu.SemaphoreType.DMA,
            pltpu.SemaphoreType.DMA,
        ],
    )
    def kge_kernel(head_hbm, rel_hbm, tail_hbm, node3_hbm, rel2_hbm,
                   out_hbm, ho, ro, to, r2i, hA, hB, rA, rB, tA, tB,
                   out_v, semA, semB):
        wid = lax.axis_index("s") * NC + lax.axis_index("c")
        base = wid * b_w

        pltpu.sync_copy(head_hbm.at[pl.ds(base, b_w)], ho)
        pltpu.sync_copy(rel_hbm.at[pl.ds(base, b_w)], ro)
        pltpu.sync_copy(tail_hbm.at[pl.ds(base, b_w)], to)

        for s in range(b_w // L):
            cs = pl.ds(s * L, L)
            r2i[cs] = lax.shift_right_logical(ro[cs], 1)

        seven = jnp.int32(7)
        three = jnp.int32(3)
        lane_iota = lax.iota(jnp.int32, L)

        def issue(p, bufs, sem):
            hbuf, rbuf, tbuf = bufs
            hv = ho[pl.ds(p * L, L)]
            tv = to[pl.ds(p * L, L)]
            pltpu.async_copy(
                rel2_hbm.at[r2i.at[pl.ds(p * L, L)]], rbuf, sem)
            for r in range(L):
                pltpu.async_copy(
                    node3_hbm.at[lax.shift_right_logical(hv[r], three)],
                    hbuf.at[r], sem)
                pltpu.async_copy(
                    node3_hbm.at[lax.shift_right_logical(tv[r], three)],
                    tbuf.at[r], sem)

        def drain(bufs, sem):
            hbuf, rbuf, tbuf = bufs
            pltpu.make_async_copy(node3_hbm.at[pl.ds(0, L)], hbuf, sem).wait()
            pltpu.make_async_copy(node3_hbm.at[pl.ds(0, L)], tbuf, sem).wait()
            pltpu.make_async_copy(rel2_hbm.at[pl.ds(0, L)], rbuf, sem).wait()

        def compute(p, bufs):
            hbuf, rbuf, tbuf = bufs
            hv = ho[pl.ds(p * L, L)]
            tv = to[pl.ds(p * L, L)]
            rp = (ro[pl.ds(p * L, L)] & jnp.int32(1)).astype(jnp.float32)
            tot = None
            for r in range(L):
                hsub = hv[r] & seven
                tsub = tv[r] & seven
                rm = _lane_shuffle(rp, jnp.full((L,), r, jnp.int32))
                acc = None
                for c in range(d_chunks):
                    cs = pl.ds(c * L, L)
                    hs = pl.ds(D + c * L, L)
                    rl = rbuf[r, cs]
                    rv = rl + rm * (rbuf[r, hs] - rl)
                    d = hbuf[r, hsub, cs] + rv - tbuf[r, tsub, cs]
                    acc = d * d if acc is None else acc + d * d
                for step in (8, 4, 2, 1):
                    acc = acc + _lane_shuffle(acc, lane_iota ^ step)
                sel = lane_iota == jnp.int32(r)
                tot = acc if r == 0 else jnp.where(sel, acc, tot)
            out_v[pl.ds(p * L, L)] = _neg_sqrt(tot + jnp.float32(1e-12))

        A = (hA, rA, tA)
        Bb = (hB, rB, tB)
        issue(0, A, semA)

        def body(k, carry):
            p0 = 2 * k
            p1 = p0 + 1
            issue(p1, Bb, semB)
            drain(A, semA)
            compute(p0, A)

            @pl.when(p1 + 1 < n_phase)
            def _():
                issue(p1 + 1, A, semA)

            drain(Bb, semB)
            compute(p1, Bb)
            return carry

        lax.fori_loop(0, n_phase // 2, body, 0)
        pltpu.sync_copy(out_v, out_hbm.at[pl.ds(base, b_w)])

    return kge_kernel


def kernel(head_index, rel_type, tail_index, node_emb, rel_emb):
    B = head_index.shape[0]
    D = node_emb.shape[1]
    info = plsc.get_sparse_core_info()
    k = _make_kernel(B, D, info.num_cores, info.num_subcores)
    node3 = node_emb.reshape(-1, SUB, D)
    rel2 = rel_emb.reshape(-1, 2 * D)
    return k(head_index.astype(jnp.int32), rel_type.astype(jnp.int32),
             tail_index.astype(jnp.int32), node3, rel2)
